# Initial kernel scaffold; baseline (speedup 1.0000x reference)
import jax
import jax.numpy as jnp
from jax.experimental import pallas as pl


def kernel(x, edge_index, edge_weight, W1, b1, W2, b2):
    raise NotImplementedError("write your pallas kernel here")


# speedup vs baseline: 8.7909x; 8.7909x over previous
import functools

import jax
import jax.numpy as jnp
from jax import lax
from jax.experimental import pallas as pl
from jax.experimental.pallas import tpu as pltpu
from jax.experimental.pallas import tpu_sc as plsc

N_NODES = 10000
N_EDGES = 320000
D = 128

NC = 2
NS = 16
NW = NC * NS

CH = 128
NCHUNKS = N_EDGES // CH
NJ = (NCHUNKS + NW - 1) // NW

RPT = 632
NPAD = NS * RPT

DW = 16


def _mesh():
    return plsc.VectorSubcoreMesh(core_axis_name="c", subcore_axis_name="s")


@functools.partial(
    pl.kernel,
    out_type=jax.ShapeDtypeStruct((NC, NPAD, DW), jnp.float32),
    mesh=_mesh(),
    compiler_params=pltpu.CompilerParams(needs_layout_passes=False),
    scratch_types=[
        pltpu.VMEM((CH,), jnp.int32),
        pltpu.VMEM((CH,), jnp.float32),
        pltpu.VMEM((CH, DW), jnp.float32),
        pltpu.VMEM_SHARED((NPAD, DW), jnp.float32),
        pltpu.SemaphoreType.DMA,
    ],
)
def _sc_deg(dst_hbm, w_hbm, out_hbm, dstv, wv, rows, acc, sem):
    c = lax.axis_index("c")
    s = lax.axis_index("s")
    wid = s * NC + c

    def zrow(i, carry):
        rows[i, :] = jnp.zeros((DW,), jnp.float32)
        return carry

    lax.fori_loop(0, CH, zrow, 0)
    for k in range(4):
        pltpu.sync_copy(rows, acc.at[pl.ds(s * RPT + k * CH, CH)])
    pltpu.sync_copy(rows.at[pl.ds(0, RPT - 4 * CH)],
                    acc.at[pl.ds(s * RPT + 4 * CH, RPT - 4 * CH)])
    plsc.subcore_barrier()

    def chunk(j, carry):
        cid = wid + NW * j

        @pl.when(cid < NCHUNKS)
        def _():
            base = cid * CH
            pltpu.sync_copy(dst_hbm.at[pl.ds(base, CH)], dstv)
            pltpu.sync_copy(w_hbm.at[pl.ds(base, CH)], wv)

            def bcast(i, carry2):
                b = plsc.load_gather(wv, [jnp.full((16,), i, jnp.int32)])
                rows[i, :] = b
                return carry2

            lax.fori_loop(0, CH, bcast, 0)
            pltpu.sync_copy(rows, acc.at[dstv], add=True)

        return carry

    lax.fori_loop(0, NJ, chunk, 0)
    plsc.subcore_barrier()
    pltpu.sync_copy(acc.at[pl.ds(s * RPT, RPT)],
                    out_hbm.at[c, pl.ds(s * RPT, RPT)])


@functools.partial(
    pl.kernel,
    out_type=jax.ShapeDtypeStruct((NC, NPAD, D), jnp.float32),
    mesh=_mesh(),
    compiler_params=pltpu.CompilerParams(needs_layout_passes=False),
    scratch_types=[
        pltpu.VMEM((CH,), jnp.int32),
        pltpu.VMEM((CH,), jnp.int32),
        pltpu.VMEM((CH,), jnp.float32),
        pltpu.VMEM((CH, D), jnp.float32),
        pltpu.VMEM_SHARED((NPAD, D), jnp.float32),
        pltpu.SemaphoreType.DMA,
    ],
)
def _sc_seg(y_hbm, src_hbm, dst_hbm, w_hbm, out_hbm,
            srcv, dstv, wv, rows, acc, sem):
    c = lax.axis_index("c")
    s = lax.axis_index("s")
    wid = s * NC + c

    def zrow(i, carry):
        for g in range(D // 16):
            rows[i, pl.ds(16 * g, 16)] = jnp.zeros((16,), jnp.float32)
        return carry

    lax.fori_loop(0, CH, zrow, 0)
    for k in range(4):
        pltpu.sync_copy(rows, acc.at[pl.ds(s * RPT + k * CH, CH)])
    pltpu.sync_copy(rows.at[pl.ds(0, RPT - 4 * CH)],
                    acc.at[pl.ds(s * RPT + 4 * CH, RPT - 4 * CH)])
    plsc.subcore_barrier()

    def chunk(j, carry):
        cid = wid + NW * j

        @pl.when(cid < NCHUNKS)
        def _():
            base = cid * CH
            pltpu.sync_copy(src_hbm.at[pl.ds(base, CH)], srcv)
            pltpu.sync_copy(dst_hbm.at[pl.ds(base, CH)], dstv)
            pltpu.sync_copy(w_hbm.at[pl.ds(base, CH)], wv)
            pltpu.async_copy(y_hbm.at[srcv], rows, sem).wait()

            def scale(i, carry2):
                b = plsc.load_gather(wv, [jnp.full((16,), i, jnp.int32)])
                for g in range(D // 16):
                    rows[i, pl.ds(16 * g, 16)] = rows[i, pl.ds(16 * g, 16)] * b
                return carry2

            lax.fori_loop(0, CH, scale, 0)
            pltpu.sync_copy(rows, acc.at[dstv], add=True)

        return carry

    lax.fori_loop(0, NJ, chunk, 0)
    plsc.subcore_barrier()
    pltpu.sync_copy(acc.at[pl.ds(s * RPT, RPT)],
                    out_hbm.at[c, pl.ds(s * RPT, RPT)])


_RB = 1000
_GRID = N_NODES // _RB


def _dinv_block(degp):
    deg = degp[0, :, 0] + degp[1, :, 0] + 1.0
    return lax.rsqrt(deg)


def _tc_mm_scale_body(x_ref, w_ref, degp_ref, y_ref):
    dinv = _dinv_block(degp_ref[...])
    xw = jnp.dot(x_ref[...], w_ref[...], preferred_element_type=jnp.float32)
    y_ref[...] = dinv[:, None] * xw


def _tc_mm_scale(x, W, degp):
    return pl.pallas_call(
        _tc_mm_scale_body,
        grid=(_GRID,),
        in_specs=[
            pl.BlockSpec((_RB, D), lambda i: (i, 0)),
            pl.BlockSpec((D, D), lambda i: (0, 0)),
            pl.BlockSpec((NC, _RB, DW), lambda i: (0, i, 0)),
        ],
        out_specs=pl.BlockSpec((_RB, D), lambda i: (i, 0)),
        out_shape=jax.ShapeDtypeStruct((N_NODES, D), jnp.float32),
    )(x, W, degp)


def _tc_combine_mm_body(s_ref, y_ref, degp_ref, b_ref, w_ref, out_ref):
    dinv = _dinv_block(degp_ref[...])
    h = dinv[:, None] * (s_ref[0] + s_ref[1] + y_ref[...]) + b_ref[...][None, :]
    hw = jnp.dot(h, w_ref[...], preferred_element_type=jnp.float32)
    out_ref[...] = dinv[:, None] * hw


def _tc_combine_mm(s_part, y, degp, b, W):
    return pl.pallas_call(
        _tc_combine_mm_body,
        grid=(_GRID,),
        in_specs=[
            pl.BlockSpec((NC, _RB, D), lambda i: (0, i, 0)),
            pl.BlockSpec((_RB, D), lambda i: (i, 0)),
            pl.BlockSpec((NC, _RB, DW), lambda i: (0, i, 0)),
            pl.BlockSpec((D,), lambda i: (0,)),
            pl.BlockSpec((D, D), lambda i: (0, 0)),
        ],
        out_specs=pl.BlockSpec((_RB, D), lambda i: (i, 0)),
        out_shape=jax.ShapeDtypeStruct((N_NODES, D), jnp.float32),
    )(s_part, y, degp, b, W)


def _tc_combine_body(s_ref, y_ref, degp_ref, b_ref, out_ref):
    dinv = _dinv_block(degp_ref[...])
    out_ref[...] = (dinv[:, None] * (s_ref[0] + s_ref[1] + y_ref[...])
                    + b_ref[...][None, :])


def _tc_combine(s_part, y, degp, b):
    return pl.pallas_call(
        _tc_combine_body,
        grid=(_GRID,),
        in_specs=[
            pl.BlockSpec((NC, _RB, D), lambda i: (0, i, 0)),
            pl.BlockSpec((_RB, D), lambda i: (i, 0)),
            pl.BlockSpec((NC, _RB, DW), lambda i: (0, i, 0)),
            pl.BlockSpec((D,), lambda i: (0,)),
        ],
        out_specs=pl.BlockSpec((_RB, D), lambda i: (i, 0)),
        out_shape=jax.ShapeDtypeStruct((N_NODES, D), jnp.float32),
    )(s_part, y, degp, b)


def kernel(x, edge_index, edge_weight, W1, b1, W2, b2):
    src = edge_index[0]
    dst = edge_index[1]
    degp = _sc_deg(dst, edge_weight)
    y1 = _tc_mm_scale(x, W1, degp)
    s1 = _sc_seg(y1, src, dst, edge_weight)
    y2 = _tc_combine_mm(s1, y1, degp, b1, W2)
    s2 = _sc_seg(y2, src, dst, edge_weight)
    out = _tc_combine(s2, y2, degp, b2)
    return out
